# Initial kernel scaffold; baseline (speedup 1.0000x reference)
import jax
import jax.numpy as jnp
from jax.experimental import pallas as pl


def kernel(x, table):
    raise NotImplementedError("write your pallas kernel here")


# speedup vs baseline: 1.7971x; 1.7971x over previous
import functools

import jax
import jax.numpy as jnp
from jax import lax
from jax.experimental import pallas as pl
from jax.experimental.pallas import tpu as pltpu
from jax.experimental.pallas import tpu_sc as plsc

VOCAB = 1000000
DIM = 64
BATCH = 16384
SEQ = 50
TOTAL = BATCH * SEQ

NC = 2
NS = 16
NW = NC * NS
PER_W = TOTAL // NW

GROUP = 128
GPC = 4
CHUNK = GPC * GROUP
N_CHUNKS = PER_W // CHUNK
GROUPS_TOTAL = TOTAL // GROUP
GROUPS_PER_W = PER_W // GROUP

_mesh = plsc.VectorSubcoreMesh(core_axis_name="c", subcore_axis_name="s")


@functools.partial(
    pl.kernel,
    mesh=_mesh,
    compiler_params=pltpu.CompilerParams(use_tc_tiling_on_sc=False),
    out_type=jax.ShapeDtypeStruct((TOTAL, DIM), jnp.float32),
    scratch_types=[
        pltpu.VMEM((GPC, GROUP), jnp.int32),
        pltpu.VMEM((CHUNK, DIM), jnp.float32),
        pltpu.SemaphoreType.DMA,
    ],
)
def _gather(x_hbm, table_hbm, out_hbm, idx_v, rows_v, sem):
    wid = lax.axis_index("s") * NC + lax.axis_index("c")
    base_g = wid * GROUPS_PER_W

    def body(c, carry):
        g0 = pl.multiple_of(base_g + c * GPC, GPC)
        pltpu.sync_copy(x_hbm.at[pl.ds(g0, GPC)], idx_v)
        copies = [
            pltpu.async_copy(
                table_hbm.at[idx_v.at[j]],
                rows_v.at[pl.ds(j * GROUP, GROUP)],
                sem,
            )
            for j in range(GPC)
        ]
        for cp in copies:
            cp.wait()
        pltpu.sync_copy(rows_v, out_hbm.at[pl.ds(g0 * GROUP, CHUNK)])
        return carry

    lax.fori_loop(0, N_CHUNKS, body, 0)


def kernel(x, table):
    x_groups = x.reshape(GROUPS_TOTAL, GROUP)
    out = _gather(x_groups, table)
    return out.reshape(BATCH, SEQ, DIM)


# speedup vs baseline: 1.8761x; 1.0439x over previous
import functools

import jax
import jax.numpy as jnp
from jax import lax
from jax.experimental import pallas as pl
from jax.experimental.pallas import tpu as pltpu
from jax.experimental.pallas import tpu_sc as plsc

VOCAB = 1000000
DIM = 64
BATCH = 16384
SEQ = 50
TOTAL = BATCH * SEQ

NC = 2
NS = 16
NW = NC * NS
PER_W = TOTAL // NW

GROUP = 128
GPC = 2
CHUNK = GPC * GROUP
N_CHUNKS = PER_W // CHUNK
NBUF = 4
N_ROUNDS = N_CHUNKS // NBUF
GROUPS_TOTAL = TOTAL // GROUP
GROUPS_PER_W = PER_W // GROUP

_mesh = plsc.VectorSubcoreMesh(core_axis_name="c", subcore_axis_name="s")


@functools.partial(
    pl.kernel,
    mesh=_mesh,
    compiler_params=pltpu.CompilerParams(use_tc_tiling_on_sc=False),
    out_type=jax.ShapeDtypeStruct((TOTAL, DIM), jnp.float32),
    scratch_types=[
        pltpu.VMEM((NBUF, GPC, GROUP), jnp.int32),
        pltpu.VMEM((NBUF, CHUNK, DIM), jnp.float32),
        pltpu.SemaphoreType.DMA((NBUF,)),
        pltpu.SemaphoreType.DMA((NBUF,)),
        pltpu.SemaphoreType.DMA((NBUF,)),
    ],
)
def _gather(x_hbm, table_hbm, out_hbm, idx_v, rows_v, s_idx, s_g, s_wb):
    wid = lax.axis_index("s") * NC + lax.axis_index("c")
    base_g = wid * GROUPS_PER_W
    base_r = base_g * GROUP

    def load_idx(c, b):
        g0 = pl.multiple_of(base_g + c * GPC, GPC)
        pltpu.async_copy(x_hbm.at[pl.ds(g0, GPC)], idx_v.at[b], s_idx.at[b])

    def wait_idx(b):
        pltpu.make_async_copy(
            x_hbm.at[pl.ds(0, GPC)], idx_v.at[b], s_idx.at[b]).wait()

    def fire_gather(b):
        for g in range(GPC):
            pltpu.async_copy(
                table_hbm.at[idx_v.at[b, g]],
                rows_v.at[b, pl.ds(g * GROUP, GROUP)],
                s_g.at[b],
            )

    def wait_gather(b):
        pltpu.make_async_copy(
            table_hbm.at[pl.ds(0, CHUNK)], rows_v.at[b], s_g.at[b]).wait()

    def start_wb(c, b):
        r0 = pl.multiple_of(base_r + c * CHUNK, CHUNK)
        pltpu.async_copy(rows_v.at[b], out_hbm.at[pl.ds(r0, CHUNK)], s_wb.at[b])

    def wait_wb(b):
        pltpu.make_async_copy(
            rows_v.at[b], out_hbm.at[pl.ds(0, CHUNK)], s_wb.at[b]).wait()

    def round_body(base, first=False, last=False):
        for j in range(NBUF):
            c = base + j
            bf = (j + NBUF - 1) % NBUF
            if not (last and j > 0):
                if not (first and j == 0):
                    wait_wb(bf)
                wait_idx(bf)
                fire_gather(bf)
            wait_gather(j)
            start_wb(c, j)
            if not last:
                load_idx(c + NBUF, j)

    for j in range(NBUF):
        load_idx(j, j)
    for j in range(NBUF - 1):
        wait_idx(j)
        fire_gather(j)
    round_body(0, first=True)

    def body(r, carry):
        round_body(r * NBUF)
        return carry

    lax.fori_loop(1, N_ROUNDS - 1, body, 0)

    round_body((N_ROUNDS - 1) * NBUF, last=True)
    for j in range(NBUF):
        wait_wb(j)


def kernel(x, table):
    x_groups = x.reshape(GROUPS_TOTAL, GROUP)
    out = _gather(x_groups, table)
    return out.reshape(BATCH, SEQ, DIM)
